# baseline (speedup 1.0000x reference)
import functools

import jax
import jax.numpy as jnp
from jax import lax
from jax.experimental import pallas as pl
from jax.experimental.pallas import tpu as pltpu
from jax.experimental.pallas import tpu_sc as plsc

D_MODEL = 64


@jax.jit
def _embedding_lookup(idx, table):
    nw_in, bpw, B1 = idx.shape
    B0 = nw_in * bpw
    info = plsc.get_sparse_core_info()
    nw = info.num_cores * info.num_subcores
    assert nw_in == nw and bpw % 2 == 0
    n_pairs = bpw // 2

    mesh = plsc.VectorSubcoreMesh(core_axis_name="c", subcore_axis_name="s")

    @functools.partial(
        pl.kernel,
        mesh=mesh,
        out_type=jax.ShapeDtypeStruct((B0, B1, D_MODEL), jnp.float32),
        scratch_types=[
            pltpu.VMEM((bpw, B1), jnp.int32),
            pltpu.VMEM((B1, D_MODEL), jnp.float32),
            pltpu.VMEM((B1, D_MODEL), jnp.float32),
            pltpu.SemaphoreType.DMA,
            pltpu.SemaphoreType.DMA,
            pltpu.SemaphoreType.DMA,
            pltpu.SemaphoreType.DMA,
        ],
        compiler_params=pltpu.CompilerParams(use_tc_tiling_on_sc=False),
    )
    def k(table_hbm, idx_hbm, out_hbm, idx_v, rows0, rows1, g0s, g1s, o0s, o1s):
        wid = lax.axis_index("s") * info.num_cores + lax.axis_index("c")
        base = wid * bpw

        def gat(g, rows, sem):
            return pltpu.make_async_copy(table_hbm.at[idx_v.at[g]], rows, sem)

        def put(g, rows, sem):
            return pltpu.make_async_copy(rows, out_hbm.at[base + g], sem)

        pltpu.sync_copy(idx_hbm.at[wid], idx_v)
        gat(0, rows0, g0s).start()

        def body(j, carry):
            g0 = 2 * j
            g1 = g0 + 1

            @pl.when(j > 0)
            def _():
                put(g0 - 1, rows1, o1s).wait()

            gat(g1, rows1, g1s).start()
            gat(g0, rows0, g0s).wait()
            put(g0, rows0, o0s).start()

            @pl.when(j < n_pairs - 1)
            def _():
                put(g0, rows0, o0s).wait()
                gat(g0 + 2, rows0, g0s).start()

            gat(g1, rows1, g1s).wait()
            put(g1, rows1, o1s).start()
            return carry

        lax.fori_loop(0, n_pairs, body, 0)
        put(bpw - 2, rows0, o0s).wait()
        put(bpw - 1, rows1, o1s).wait()

    return k(table, idx)


def kernel(x, table):
    info = plsc.get_sparse_core_info()
    nw = info.num_cores * info.num_subcores
    idx = x.reshape(nw, -1, x.shape[1])
    return _embedding_lookup(idx, table)


# speedup vs baseline: 1.0888x; 1.0888x over previous
import functools

import jax
import jax.numpy as jnp
from jax import lax
from jax.experimental import pallas as pl
from jax.experimental.pallas import tpu as pltpu
from jax.experimental.pallas import tpu_sc as plsc

D = 64
B0, B1 = 4096, 200


@jax.jit
def _embedding_lookup(idx_flat, t128):
    info = plsc.get_sparse_core_info()
    nc = info.num_cores
    nw = nc * info.num_subcores
    bpw = B0 // nw

    mesh = plsc.VectorSubcoreMesh(core_axis_name="c", subcore_axis_name="s")

    @functools.partial(
        pl.kernel,
        mesh=mesh,
        out_type=jax.ShapeDtypeStruct((B0, B1, D), jnp.float32),
        scratch_types=[
            pltpu.VMEM((B1,), jnp.int32),
            pltpu.VMEM((B1,), jnp.int32),
            pltpu.VMEM((B1, 128), jnp.float32),
            pltpu.VMEM((B1, 128), jnp.float32),
            pltpu.VMEM((B1, D), jnp.float32),
            pltpu.SemaphoreType.DMA,
            pltpu.SemaphoreType.DMA,
        ],
        compiler_params=pltpu.CompilerParams(
            use_tc_tiling_on_sc=True, needs_layout_passes=False),
    )
    def gather(t128_hbm, idx_hbm, out_hbm, idx0, idx1, g0, g1, b64, s0, s1):
        wid = lax.axis_index("s") * nc + lax.axis_index("c")
        base = wid * bpw
        idx_b = (idx0, idx1)
        g_b = (g0, g1)
        s_b = (s0, s1)

        def start(j, p):
            pltpu.sync_copy(idx_hbm.at[pl.ds((base + j) * B1, B1)], idx_b[p])
            pltpu.make_async_copy(t128_hbm.at[idx_b[p]], g_b[p], s_b[p]).start()

        def finish(j, p):
            pltpu.make_async_copy(t128_hbm.at[idx_b[p]], g_b[p], s_b[p]).wait()
            src = g_b[p]

            def rows(u, c2):
                for rr in range(8):
                    i = u * 8 + rr
                    for ch in range(D // 16):
                        b64[i, pl.ds(ch * 16, 16)] = src[i, pl.ds(ch * 16, 16)]
                return c2

            lax.fori_loop(0, B1 // 8, rows, 0)
            pltpu.sync_copy(b64, out_hbm.at[base + j])

        start(0, 0)

        def body(h, c):
            j0 = 2 * h
            start(j0 + 1, 1)
            finish(j0, 0)

            @pl.when(h < bpw // 2 - 1)
            def _():
                start(j0 + 2, 0)

            finish(j0 + 1, 1)
            return c

        lax.fori_loop(0, bpw // 2, body, 0)

    return gather(t128, idx_flat)


def kernel(x, table):
    t128 = jnp.pad(table, ((0, 0), (0, 128 - D)))
    return _embedding_lookup(x.reshape(-1), t128)
